# baseline (speedup 1.0000x reference)
import functools

import jax
import jax.numpy as jnp
from jax.experimental import pallas as pl
from jax.experimental.pallas import tpu as pltpu


def _adjacency(e_ref, N):
    f32 = jnp.float32
    e = e_ref[...]
    erow = e[0:1, :]
    ecol = e[1:2, :]
    Epad = e.shape[1]
    nio = jax.lax.broadcasted_iota(jnp.int32, (N, Epad), 0).astype(f32)
    Oc = (ecol == nio).astype(f32)
    Or = (erow == nio).astype(f32)
    A = jax.lax.dot_general(Oc, Or, (((1,), (1,)), ((), ())),
                            preferred_element_type=f32)
    ri = jax.lax.broadcasted_iota(jnp.int32, (N, N), 0)
    ci = jax.lax.broadcasted_iota(jnp.int32, (N, N), 1)
    eye = (ri == ci).astype(f32)
    Ah = A + eye
    deg = jnp.sum(Ah, axis=1, keepdims=True)
    dinv = jax.lax.rsqrt(deg)
    D = eye * dinv
    return jnp.dot(jnp.dot(D, Ah, preferred_element_type=f32), D,
                   preferred_element_type=f32)


def _fused_step(x_ref, e_ref, w1_ref, b1_ref, w2_ref, b2_ref,
                wih_ref, whh_ref, bias_ref, fcw_ref, fcb_ref,
                out_ref, h_ref, c_ref, m_ref, *, N, NF, B, H, LH, W, CH):
    w = pl.program_id(0)
    f32 = jnp.float32

    @pl.when(w == 0)
    def _init():
        h_ref[...] = jnp.zeros_like(h_ref)
        c_ref[...] = jnp.zeros_like(c_ref)
        m_ref[...] = jnp.full_like(m_ref, -jnp.inf)

    bf16 = jnp.bfloat16
    An = _adjacency(e_ref, N)
    Anb = An.astype(bf16)
    w1 = w1_ref[...].astype(bf16)
    w2 = w2_ref[...]
    b1 = b1_ref[...].astype(bf16)
    b2t = b2_ref[...]

    for t in range(CH):
        x = x_ref[t]
        ax = jnp.dot(An, x, preferred_element_type=f32)
        axb = ax.astype(bf16)
        acc = (jax.lax.broadcast_in_dim(axb[:, 0:B], (N, B, H), (0, 1))
               * w1[0].reshape(1, 1, H))
        for k in range(1, NF):
            acc = acc + (jax.lax.broadcast_in_dim(axb[:, k * B:(k + 1) * B],
                                                  (N, B, H), (0, 1))
                         * w1[k].reshape(1, 1, H))
        xw1 = acc.reshape(N * B, H)
        h1 = jnp.maximum(xw1 + b1, 0)
        xw2 = jnp.dot(h1, w2, preferred_element_type=f32).astype(bf16)
        t2 = jnp.dot(Anb, xw2.reshape(N, B * H),
                     preferred_element_type=f32)
        h2 = jnp.maximum(t2 + b2t, 0)
        pooled = (jnp.sum(h2, axis=0, keepdims=True) * (1.0 / N)
                  ).astype(bf16).reshape(B, H)

        gates = (jnp.dot(pooled, wih_ref[...],
                         preferred_element_type=f32)
                 + jnp.dot(h_ref[...].astype(bf16), whh_ref[...],
                           preferred_element_type=f32)
                 + bias_ref[...])
        sig = jax.nn.sigmoid(gates[:, 0:3 * LH])
        gg = jnp.tanh(gates[:, 3 * LH:4 * LH])
        ig = sig[:, 0:LH]
        fg = sig[:, LH:2 * LH]
        og = sig[:, 2 * LH:3 * LH]
        c_new = fg * c_ref[...] + ig * gg
        h_new = og * jnp.tanh(c_new)
        c_ref[...] = c_new
        h_ref[...] = h_new
        m_ref[...] = jnp.maximum(m_ref[...], h_new)

    @pl.when(w == (W // CH) - 1)
    def _final():
        logits = jnp.dot(m_ref[...], fcw_ref[...],
                         preferred_element_type=f32) + fcb_ref[...]
        out_ref[...] = jax.nn.sigmoid(logits)


def _pick_chunk(W, cands):
    for c in cands:
        if W % c == 0:
            return c
    return 1


def kernel(all_keypoint_batches, edge_index, conv1_W, conv1_b, conv2_W,
           conv2_b, W_ih, W_hh, b_ih, b_hh, fc_W, fc_b):
    B, W, N, NF = all_keypoint_batches.shape
    H = conv1_W.shape[1]
    LH = W_hh.shape[1]
    NC = fc_W.shape[0]
    E = edge_index.shape[1]
    CH = _pick_chunk(W, (5, 4, 2))

    Xn = jnp.transpose(all_keypoint_batches, (1, 2, 3, 0)).reshape(W, N, NF * B)

    Epad = max(32, ((E + 31) // 32) * 32)
    epad = jnp.full((8, Epad), -1.0, dtype=jnp.float32)
    epad = epad.at[:2, :E].set(edge_index.astype(jnp.float32))

    wih = W_ih.T
    whh = W_hh.T
    bias = (b_ih + b_hh).reshape(1, 4 * LH)
    perm = jnp.concatenate([jnp.arange(0, 2 * LH),
                            jnp.arange(3 * LH, 4 * LH),
                            jnp.arange(2 * LH, 3 * LH)])
    wih = wih[:, perm].astype(jnp.bfloat16)
    whh = whh[:, perm].astype(jnp.bfloat16)
    bias = bias[:, perm]
    b1 = conv1_b.reshape(1, H)
    b2 = jnp.tile(conv2_b, B).reshape(1, B * H)
    fcw = fc_W.T
    fcb = fc_b.reshape(1, NC)

    full = lambda shape: pl.BlockSpec(shape, lambda w: (0,) * len(shape))

    out = pl.pallas_call(
        functools.partial(_fused_step, N=N, NF=NF, B=B, H=H, LH=LH, W=W,
                          CH=CH),
        grid=(W // CH,),
        in_specs=[
            pl.BlockSpec((CH, N, B * NF), lambda w: (w, 0, 0)),
            full((8, Epad)),
            full((NF, H)),
            full((1, H)),
            full((H, H)),
            full((1, B * H)),
            full((H, 4 * LH)),
            full((LH, 4 * LH)),
            full((1, 4 * LH)),
            full((LH, NC)),
            full((1, NC)),
        ],
        out_specs=pl.BlockSpec((B, NC), lambda w: (0, 0)),
        out_shape=jax.ShapeDtypeStruct((B, NC), jnp.float32),
        scratch_shapes=[
            pltpu.VMEM((B, LH), jnp.float32),
            pltpu.VMEM((B, LH), jnp.float32),
            pltpu.VMEM((B, LH), jnp.float32),
        ],
        compiler_params=pltpu.CompilerParams(
            dimension_semantics=("arbitrary",)),
    )(Xn, epad, conv1_W, b1, conv2_W.astype(jnp.bfloat16), b2, wih, whh,
      bias, fcw, fcb)
    return out[:, 0]


# speedup vs baseline: 1.0179x; 1.0179x over previous
import functools

import jax
import jax.numpy as jnp
from jax.experimental import pallas as pl
from jax.experimental.pallas import tpu as pltpu


def _adjacency(e_ref, N):
    f32 = jnp.float32
    e = e_ref[...]
    erow = e[0:1, :]
    ecol = e[1:2, :]
    Epad = e.shape[1]
    nio = jax.lax.broadcasted_iota(jnp.int32, (N, Epad), 0).astype(f32)
    Oc = (ecol == nio).astype(f32)
    Or = (erow == nio).astype(f32)
    A = jax.lax.dot_general(Oc, Or, (((1,), (1,)), ((), ())),
                            preferred_element_type=f32)
    ri = jax.lax.broadcasted_iota(jnp.int32, (N, N), 0)
    ci = jax.lax.broadcasted_iota(jnp.int32, (N, N), 1)
    eye = (ri == ci).astype(f32)
    Ah = A + eye
    deg = jnp.sum(Ah, axis=1, keepdims=True)
    dinv = jax.lax.rsqrt(deg)
    D = eye * dinv
    return jnp.dot(jnp.dot(D, Ah, preferred_element_type=f32), D,
                   preferred_element_type=f32)


def _fused_step(x_ref, e_ref, w1_ref, b1_ref, w2_ref, b2_ref,
                wih_ref, whh_ref, bias_ref, fcw_ref, fcb_ref,
                out_ref, h_ref, c_ref, m_ref, *, N, NF, B, H, LH, W, CH):
    w = pl.program_id(0)
    f32 = jnp.float32

    @pl.when(w == 0)
    def _init():
        h_ref[...] = jnp.zeros_like(h_ref)
        c_ref[...] = jnp.zeros_like(c_ref)
        m_ref[...] = jnp.full_like(m_ref, -jnp.inf)

    bf16 = jnp.bfloat16
    An = _adjacency(e_ref, N)
    Anb = An.astype(bf16)
    w1 = w1_ref[...].astype(bf16)
    w2 = w2_ref[...]
    b1 = b1_ref[...].astype(bf16)
    b2t = b2_ref[...]

    for t in range(CH):
        x = x_ref[t]
        ax = jnp.dot(An, x, preferred_element_type=f32)
        axb = ax.astype(bf16)
        acc = (jax.lax.broadcast_in_dim(axb[:, 0:B], (N, B, H), (0, 1))
               * w1[0].reshape(1, 1, H))
        for k in range(1, NF):
            acc = acc + (jax.lax.broadcast_in_dim(axb[:, k * B:(k + 1) * B],
                                                  (N, B, H), (0, 1))
                         * w1[k].reshape(1, 1, H))
        xw1 = acc.reshape(N * B, H)
        h1 = jnp.maximum(xw1 + b1, 0)
        xw2 = jnp.dot(h1, w2, preferred_element_type=f32).astype(bf16)
        t2 = jnp.dot(Anb, xw2.reshape(N, B * H),
                     preferred_element_type=f32)
        h2 = jnp.maximum(t2 + b2t, 0)
        pooled = (jnp.sum(h2, axis=0, keepdims=True) * (1.0 / N)
                  ).astype(bf16).reshape(B, H)

        gates = (jnp.dot(pooled, wih_ref[...],
                         preferred_element_type=f32)
                 + jnp.dot(h_ref[...].astype(bf16), whh_ref[...],
                           preferred_element_type=f32)
                 + bias_ref[...])
        sig = jax.nn.sigmoid(gates[:, 0:3 * LH])
        gg = jnp.tanh(gates[:, 3 * LH:4 * LH])
        ig = sig[:, 0:LH]
        fg = sig[:, LH:2 * LH]
        og = sig[:, 2 * LH:3 * LH]
        c_new = fg * c_ref[...] + ig * gg
        h_new = og * jnp.tanh(c_new)
        c_ref[...] = c_new
        h_ref[...] = h_new
        m_ref[...] = jnp.maximum(m_ref[...], h_new)

    @pl.when(w == (W // CH) - 1)
    def _final():
        logits = jnp.dot(m_ref[...], fcw_ref[...],
                         preferred_element_type=f32) + fcb_ref[...]
        out_ref[...] = jax.nn.sigmoid(logits)


def _pick_chunk(W, cands):
    for c in cands:
        if W % c == 0:
            return c
    return 1


def kernel(all_keypoint_batches, edge_index, conv1_W, conv1_b, conv2_W,
           conv2_b, W_ih, W_hh, b_ih, b_hh, fc_W, fc_b):
    B, W, N, NF = all_keypoint_batches.shape
    H = conv1_W.shape[1]
    LH = W_hh.shape[1]
    NC = fc_W.shape[0]
    E = edge_index.shape[1]
    CH = _pick_chunk(W, (10, 8, 5, 4, 2))

    Xn = jnp.transpose(all_keypoint_batches, (1, 2, 3, 0)).reshape(W, N, NF * B)

    Epad = max(32, ((E + 31) // 32) * 32)
    epad = jnp.full((8, Epad), -1.0, dtype=jnp.float32)
    epad = epad.at[:2, :E].set(edge_index.astype(jnp.float32))

    wih = W_ih.T
    whh = W_hh.T
    bias = (b_ih + b_hh).reshape(1, 4 * LH)
    perm = jnp.concatenate([jnp.arange(0, 2 * LH),
                            jnp.arange(3 * LH, 4 * LH),
                            jnp.arange(2 * LH, 3 * LH)])
    wih = wih[:, perm].astype(jnp.bfloat16)
    whh = whh[:, perm].astype(jnp.bfloat16)
    bias = bias[:, perm]
    b1 = conv1_b.reshape(1, H)
    b2 = jnp.tile(conv2_b, B).reshape(1, B * H)
    fcw = fc_W.T
    fcb = fc_b.reshape(1, NC)

    full = lambda shape: pl.BlockSpec(shape, lambda w: (0,) * len(shape))

    out = pl.pallas_call(
        functools.partial(_fused_step, N=N, NF=NF, B=B, H=H, LH=LH, W=W,
                          CH=CH),
        grid=(W // CH,),
        in_specs=[
            pl.BlockSpec((CH, N, B * NF), lambda w: (w, 0, 0)),
            full((8, Epad)),
            full((NF, H)),
            full((1, H)),
            full((H, H)),
            full((1, B * H)),
            full((H, 4 * LH)),
            full((LH, 4 * LH)),
            full((1, 4 * LH)),
            full((LH, NC)),
            full((1, NC)),
        ],
        out_specs=pl.BlockSpec((B, NC), lambda w: (0, 0)),
        out_shape=jax.ShapeDtypeStruct((B, NC), jnp.float32),
        scratch_shapes=[
            pltpu.VMEM((B, LH), jnp.float32),
            pltpu.VMEM((B, LH), jnp.float32),
            pltpu.VMEM((B, LH), jnp.float32),
        ],
        compiler_params=pltpu.CompilerParams(
            dimension_semantics=("arbitrary",)),
    )(Xn, epad, conv1_W, b1, conv2_W.astype(jnp.bfloat16), b2, wih, whh,
      bias, fcw, fcb)
    return out[:, 0]
